# baseline (speedup 1.0000x reference)
import numpy as np
import jax
import jax.numpy as jnp
from jax.experimental import pallas as pl
from jax.experimental.pallas import tpu as pltpu

BN = 512


def _band(w_len, x_len, dx, pc):
    e = np.zeros((w_len, x_len), np.float32)
    for xh in range(x_len):
        w = 2 * xh + pc + dx - 1
        if 0 <= w < w_len:
            e[w, xh] = 1.0
    return e


_E1 = [[_band(32, 16, dx, pc) for pc in (0, 1)] for dx in range(3)]
_E2 = [[_band(16, 8, dx, pc) for pc in (0, 1)] for dx in range(3)]


def _fused_kernel(x_ref, w1c_ref, b1c_ref, w2c_ref, b2c_ref,
                  wf1v_ref, bf1_ref, wf2_ref, bf2_ref, o_ref):
    n = x_ref.shape[0]

    zrow1 = jnp.zeros((n, 1, 96), jnp.bfloat16)
    xs = [x_ref[:, ci].reshape(n * 8, 128) for ci in range(3)]
    base = []
    for r in range(4):
        piece = jnp.concatenate([xs[ci][:, 32 * r:32 * r + 32]
                                 for ci in range(3)], axis=1)
        base.append(piece.astype(jnp.bfloat16).reshape(n, 8, 96))
    xm = jnp.concatenate([zrow1, base[3][:, 0:7, :]], axis=1)
    xp = jnp.concatenate([base[0][:, 1:8, :], zrow1], axis=1)

    def x_rows(c):
        r, k = c % 4, (c - c % 4) // 4
        v = base[r] if k == 0 else (xm if k < 0 else xp)
        return v.reshape(n * 8, 96)

    zrow2 = jnp.zeros((n, 1, 256), jnp.bfloat16)
    a1 = {}
    for q in (0, 1):
        m = None
        for p in (0, 1):
            for pc in (0, 1):
                acc = jnp.zeros((n * 8, 256), jnp.float32)
                for dy in range(3):
                    acc = acc + jnp.dot(x_rows(2 * q + p - 1 + dy),
                                        w1c_ref[dy, pc],
                                        preferred_element_type=jnp.float32)
                m = acc if m is None else jnp.maximum(m, acc)
        a1q = jnp.maximum(m + b1c_ref[...], 0.0).reshape(n, 8, 256)
        a1[q] = a1q.astype(jnp.bfloat16)
    a1m = jnp.concatenate([zrow2, a1[1][:, 0:7, :]], axis=1)
    a1p = jnp.concatenate([a1[0][:, 1:8, :], zrow2], axis=1)

    a1v = {(1, -1): a1m, (0, 0): a1[0], (1, 0): a1[1], (0, 1): a1p}
    m2 = None
    for p2 in (0, 1):
        for pc in (0, 1):
            acc = jnp.zeros((n * 8, 256), jnp.float32)
            for dy in range(3):
                c2 = p2 - 1 + dy
                rows = a1v[(c2 % 2, (c2 - c2 % 2) // 2)].reshape(n * 8, 256)
                acc = acc + jnp.dot(rows, w2c_ref[dy, pc],
                                    preferred_element_type=jnp.float32)
            m2 = acc if m2 is None else jnp.maximum(m2, acc)
    a2 = jnp.maximum(m2 + b2c_ref[...], 0.0).reshape(n, 8, 256)
    a2 = a2.astype(jnp.bfloat16)

    h = jnp.zeros((n, 256), jnp.float32)
    for s in range(8):
        h = h + jnp.dot(a2[:, s, :], wf1v_ref[s],
                        preferred_element_type=jnp.float32)
    h = jnp.maximum(h + bf1_ref[...], 0.0)
    o_ref[...] = (jnp.dot(h.astype(jnp.bfloat16), wf2_ref[...],
                          preferred_element_type=jnp.float32)
                  + bf2_ref[...])


@jax.jit
def _forward(x_nchw, w1, b1, w2, b2, wf1, bf1, wf2, bf2):
    B = x_nchw.shape[0]
    xr = x_nchw.astype(jnp.float32).reshape(B, 3, 8, 128)

    w1c = jnp.stack([jnp.stack([sum(jnp.kron(w1[dy * 3 + dx], jnp.asarray(_E1[dx][pc]))
                                    for dx in range(3)) for pc in (0, 1)])
                     for dy in range(3)]).astype(jnp.bfloat16)
    w2c = jnp.stack([jnp.stack([sum(jnp.kron(w2[dy * 3 + dx], jnp.asarray(_E2[dx][pc]))
                                    for dx in range(3)) for pc in (0, 1)])
                     for dy in range(3)]).astype(jnp.bfloat16)
    b1c = jnp.repeat(b1, 16, axis=1)
    b2c = jnp.repeat(b2, 8, axis=1)
    wf1v = jnp.transpose(wf1.reshape(8, 8, 32, 256), (0, 2, 1, 3)).reshape(8, 256, 256).astype(jnp.bfloat16)

    return pl.pallas_call(
        _fused_kernel,
        out_shape=jax.ShapeDtypeStruct((B, 50), jnp.float32),
        grid=(B // BN,),
        in_specs=[
            pl.BlockSpec((BN, 3, 8, 128), lambda b: (b, 0, 0, 0)),
            pl.BlockSpec((3, 2, 96, 256), lambda b: (0, 0, 0, 0)),
            pl.BlockSpec((1, 256), lambda b: (0, 0)),
            pl.BlockSpec((3, 2, 256, 256), lambda b: (0, 0, 0, 0)),
            pl.BlockSpec((1, 256), lambda b: (0, 0)),
            pl.BlockSpec((8, 256, 256), lambda b: (0, 0, 0)),
            pl.BlockSpec((1, 256), lambda b: (0, 0)),
            pl.BlockSpec((256, 50), lambda b: (0, 0)),
            pl.BlockSpec((1, 50), lambda b: (0, 0)),
        ],
        out_specs=pl.BlockSpec((BN, 50), lambda b: (b, 0)),
        compiler_params=pltpu.CompilerParams(
            dimension_semantics=("arbitrary",)),
    )(xr, w1c, b1c, w2c, b2c, wf1v, bf1, wf2.astype(jnp.bfloat16), bf2)


def kernel(x_nchw, w1, b1, w2, b2, wf1, bf1, wf2, bf2):
    return _forward(x_nchw, w1, b1, w2, b2, wf1, bf1, wf2, bf2)


# speedup vs baseline: 1.1164x; 1.1164x over previous
import numpy as np
import jax
import jax.numpy as jnp
from jax.experimental import pallas as pl
from jax.experimental.pallas import tpu as pltpu

BN = 256


def _band(w_len, x_len, dx, pc):
    e = np.zeros((w_len, x_len), np.float32)
    for xh in range(x_len):
        w = 2 * xh + pc + dx - 1
        if 0 <= w < w_len:
            e[w, xh] = 1.0
    return e


_E1 = [[_band(32, 16, dx, pc) for pc in (0, 1)] for dx in range(3)]
_E2 = [[_band(16, 8, dx, pc) for pc in (0, 1)] for dx in range(3)]


def _fused_kernel(x_ref, w1c_ref, b1c_ref, w2c_ref, b2c_ref,
                  wf1v_ref, bf1_ref, wf2_ref, bf2_ref, o_ref):
    n = x_ref.shape[0]

    zrow1 = jnp.zeros((n, 1, 96), jnp.bfloat16)
    xs = [x_ref[:, ci].reshape(n * 8, 128) for ci in range(3)]
    base = []
    for r in range(4):
        piece = jnp.concatenate([xs[ci][:, 32 * r:32 * r + 32]
                                 for ci in range(3)], axis=1)
        base.append(piece.astype(jnp.bfloat16).reshape(n, 8, 96))
    xm = jnp.concatenate([zrow1, base[3][:, 0:7, :]], axis=1)
    xp = jnp.concatenate([base[0][:, 1:8, :], zrow1], axis=1)

    def x_rows(c):
        r, k = c % 4, (c - c % 4) // 4
        v = base[r] if k == 0 else (xm if k < 0 else xp)
        return v.reshape(n * 8, 96)

    zrow2 = jnp.zeros((n, 1, 256), jnp.bfloat16)
    a1 = {}
    for q in (0, 1):
        m = None
        for p in (0, 1):
            for pc in (0, 1):
                acc = jnp.zeros((n * 8, 256), jnp.float32)
                for dy in range(3):
                    acc = acc + jnp.dot(x_rows(2 * q + p - 1 + dy),
                                        w1c_ref[dy, pc],
                                        preferred_element_type=jnp.float32)
                m = acc if m is None else jnp.maximum(m, acc)
        a1q = jnp.maximum(m + b1c_ref[...], 0.0).reshape(n, 8, 256)
        a1[q] = a1q.astype(jnp.bfloat16)
    a1m = jnp.concatenate([zrow2, a1[1][:, 0:7, :]], axis=1)
    a1p = jnp.concatenate([a1[0][:, 1:8, :], zrow2], axis=1)

    a1v = {(1, -1): a1m, (0, 0): a1[0], (1, 0): a1[1], (0, 1): a1p}
    m2 = None
    for p2 in (0, 1):
        for pc in (0, 1):
            acc = jnp.zeros((n * 8, 256), jnp.float32)
            for dy in range(3):
                c2 = p2 - 1 + dy
                rows = a1v[(c2 % 2, (c2 - c2 % 2) // 2)].reshape(n * 8, 256)
                acc = acc + jnp.dot(rows, w2c_ref[dy, pc],
                                    preferred_element_type=jnp.float32)
            m2 = acc if m2 is None else jnp.maximum(m2, acc)
    a2 = jnp.maximum(m2 + b2c_ref[...], 0.0).reshape(n, 8, 256)
    a2 = a2.astype(jnp.bfloat16)

    h = jnp.zeros((n, 256), jnp.float32)
    for s in range(8):
        h = h + jnp.dot(a2[:, s, :], wf1v_ref[s],
                        preferred_element_type=jnp.float32)
    h = jnp.maximum(h + bf1_ref[...], 0.0)
    o_ref[...] = (jnp.dot(h.astype(jnp.bfloat16), wf2_ref[...],
                          preferred_element_type=jnp.float32)
                  + bf2_ref[...])


@jax.jit
def _forward(x_nchw, w1, b1, w2, b2, wf1, bf1, wf2, bf2):
    B = x_nchw.shape[0]
    xr = x_nchw.astype(jnp.float32).reshape(B, 3, 8, 128)

    w1c = jnp.stack([jnp.stack([
        sum(jnp.einsum('io,wh->iwho', w1[dy * 3 + dx], jnp.asarray(_E1[dx][pc]))
            for dx in range(3)).reshape(96, 256) for pc in (0, 1)])
        for dy in range(3)]).astype(jnp.bfloat16)
    w2c = jnp.stack([jnp.stack([
        sum(jnp.einsum('io,wh->wiho', w2[dy * 3 + dx], jnp.asarray(_E2[dx][pc]))
            for dx in range(3)).reshape(256, 256) for pc in (0, 1)])
        for dy in range(3)]).astype(jnp.bfloat16)
    b1c = jnp.tile(b1, (1, 16))
    b2c = jnp.tile(b2, (1, 8))
    wf1v = wf1.reshape(8, 256, 256).astype(jnp.bfloat16)

    return pl.pallas_call(
        _fused_kernel,
        out_shape=jax.ShapeDtypeStruct((B, 50), jnp.float32),
        grid=(B // BN,),
        in_specs=[
            pl.BlockSpec((BN, 3, 8, 128), lambda b: (b, 0, 0, 0)),
            pl.BlockSpec((3, 2, 96, 256), lambda b: (0, 0, 0, 0)),
            pl.BlockSpec((1, 256), lambda b: (0, 0)),
            pl.BlockSpec((3, 2, 256, 256), lambda b: (0, 0, 0, 0)),
            pl.BlockSpec((1, 256), lambda b: (0, 0)),
            pl.BlockSpec((8, 256, 256), lambda b: (0, 0, 0)),
            pl.BlockSpec((1, 256), lambda b: (0, 0)),
            pl.BlockSpec((256, 50), lambda b: (0, 0)),
            pl.BlockSpec((1, 50), lambda b: (0, 0)),
        ],
        out_specs=pl.BlockSpec((BN, 50), lambda b: (b, 0)),
        compiler_params=pltpu.CompilerParams(
            dimension_semantics=("arbitrary",)),
    )(xr, w1c, b1c, w2c, b2c, wf1v, bf1, wf2.astype(jnp.bfloat16), bf2)


def kernel(x_nchw, w1, b1, w2, b2, wf1, bf1, wf2, bf2):
    return _forward(x_nchw, w1, b1, w2, b2, wf1, bf1, wf2, bf2)


# speedup vs baseline: 1.1196x; 1.0029x over previous
import numpy as np
import jax
import jax.numpy as jnp
from jax.experimental import pallas as pl
from jax.experimental.pallas import tpu as pltpu

BN = 256


def _band(w_len, x_len, dx, pc):
    e = np.zeros((w_len, x_len), np.float32)
    for xh in range(x_len):
        w = 2 * xh + pc + dx - 1
        if 0 <= w < w_len:
            e[w, xh] = 1.0
    return e


_E1 = [[_band(32, 16, dx, pc) for pc in (0, 1)] for dx in range(3)]
_E2 = [[_band(16, 8, dx, pc) for pc in (0, 1)] for dx in range(3)]


def _fused_kernel(x_ref, w1c_ref, b1c_ref, w2c_ref, b2c_ref,
                  wf1v_ref, bf1_ref, wf2_ref, bf2_ref, o_ref):
    n = x_ref.shape[0]

    zrow1 = jnp.zeros((n, 1, 96), jnp.bfloat16)
    xs = [x_ref[:, ci].reshape(n * 8, 128) for ci in range(3)]
    base = []
    for r in range(4):
        piece = jnp.concatenate([xs[ci][:, 32 * r:32 * r + 32]
                                 for ci in range(3)], axis=1)
        base.append(piece.astype(jnp.bfloat16).reshape(n, 8, 96))
    xm = jnp.concatenate([zrow1, base[3][:, 0:7, :]], axis=1)
    xp = jnp.concatenate([base[0][:, 1:8, :], zrow1], axis=1)

    def x_rows(c):
        r, k = c % 4, (c - c % 4) // 4
        v = base[r] if k == 0 else (xm if k < 0 else xp)
        return v.reshape(n * 8, 96)

    zrow2 = jnp.zeros((n, 1, 256), jnp.float32)
    a1 = {}
    for q in (0, 1):
        m = None
        for p in (0, 1):
            for pc in (0, 1):
                acc = jnp.zeros((n * 8, 256), jnp.float32)
                for dy in range(3):
                    acc = acc + jnp.dot(x_rows(2 * q + p - 1 + dy),
                                        w1c_ref[dy, pc],
                                        preferred_element_type=jnp.float32)
                m = acc if m is None else jnp.maximum(m, acc)
        a1q = jnp.maximum(m + b1c_ref[...], 0.0).reshape(n, 8, 256)
        a1[q] = a1q
    a1m = jnp.concatenate([zrow2, a1[1][:, 0:7, :]], axis=1)
    a1p = jnp.concatenate([a1[0][:, 1:8, :], zrow2], axis=1)

    a1v = {(1, -1): a1m, (0, 0): a1[0], (1, 0): a1[1], (0, 1): a1p}
    m2 = None
    for p2 in (0, 1):
        for pc in (0, 1):
            acc = jnp.zeros((n * 8, 256), jnp.float32)
            for dy in range(3):
                c2 = p2 - 1 + dy
                rows = a1v[(c2 % 2, (c2 - c2 % 2) // 2)].reshape(n * 8, 256)
                acc = acc + jnp.dot(rows, w2c_ref[dy, pc],
                                    preferred_element_type=jnp.float32)
            m2 = acc if m2 is None else jnp.maximum(m2, acc)
    a2 = jnp.maximum(m2 + b2c_ref[...], 0.0).reshape(n, 8, 256)


    h = jnp.zeros((n, 256), jnp.float32)
    for s in range(8):
        h = h + jnp.dot(a2[:, s, :], wf1v_ref[s],
                        preferred_element_type=jnp.float32)
    h = jnp.maximum(h + bf1_ref[...], 0.0)
    o_ref[...] = (jnp.dot(h, wf2_ref[...],
                          preferred_element_type=jnp.float32)
                  + bf2_ref[...])


@jax.jit
def _forward(x_nchw, w1, b1, w2, b2, wf1, bf1, wf2, bf2):
    B = x_nchw.shape[0]
    xr = x_nchw.astype(jnp.float32).reshape(B, 3, 8, 128)

    w1c = jnp.stack([jnp.stack([
        sum(jnp.einsum('io,wh->iwho', w1[dy * 3 + dx], jnp.asarray(_E1[dx][pc]))
            for dx in range(3)).reshape(96, 256) for pc in (0, 1)])
        for dy in range(3)]).astype(jnp.bfloat16)
    w2c = jnp.stack([jnp.stack([
        sum(jnp.einsum('io,wh->wiho', w2[dy * 3 + dx], jnp.asarray(_E2[dx][pc]))
            for dx in range(3)).reshape(256, 256) for pc in (0, 1)])
        for dy in range(3)])
    b1c = jnp.tile(b1, (1, 16))
    b2c = jnp.tile(b2, (1, 8))
    wf1v = wf1.reshape(8, 256, 256)

    return pl.pallas_call(
        _fused_kernel,
        out_shape=jax.ShapeDtypeStruct((B, 50), jnp.float32),
        grid=(B // BN,),
        in_specs=[
            pl.BlockSpec((BN, 3, 8, 128), lambda b: (b, 0, 0, 0)),
            pl.BlockSpec((3, 2, 96, 256), lambda b: (0, 0, 0, 0)),
            pl.BlockSpec((1, 256), lambda b: (0, 0)),
            pl.BlockSpec((3, 2, 256, 256), lambda b: (0, 0, 0, 0)),
            pl.BlockSpec((1, 256), lambda b: (0, 0)),
            pl.BlockSpec((8, 256, 256), lambda b: (0, 0, 0)),
            pl.BlockSpec((1, 256), lambda b: (0, 0)),
            pl.BlockSpec((256, 50), lambda b: (0, 0)),
            pl.BlockSpec((1, 50), lambda b: (0, 0)),
        ],
        out_specs=pl.BlockSpec((BN, 50), lambda b: (b, 0)),
        compiler_params=pltpu.CompilerParams(
            dimension_semantics=("arbitrary",)),
    )(xr, w1c, b1c, w2c, b2c, wf1v, bf1, wf2, bf2)


def kernel(x_nchw, w1, b1, w2, b2, wf1, bf1, wf2, bf2):
    return _forward(x_nchw, w1, b1, w2, b2, wf1, bf1, wf2, bf2)
